# baseline (speedup 1.0000x reference)
import functools

import jax
import jax.numpy as jnp
from jax import lax
from jax.experimental import pallas as pl
from jax.experimental.pallas import tpu as pltpu
from jax.experimental.pallas import tpu_sc as plsc

BATCH = 1024
N_HIDDEN = 16
N_FEATURES = 100000

_NC = 2
_NS = 16
_NW = _NC * _NS
_B_PER_W = BATCH // _NW


def _sc_gather(data, emb):
    mesh = plsc.VectorSubcoreMesh(core_axis_name="c", subcore_axis_name="s")

    @functools.partial(
        pl.kernel,
        mesh=mesh,
        out_type=jax.ShapeDtypeStruct((BATCH, N_HIDDEN), jnp.float32),
        scratch_types=[
            pltpu.VMEM((_B_PER_W,), jnp.int32),
            pltpu.VMEM((_B_PER_W, N_HIDDEN), jnp.float32),
            pltpu.SemaphoreType.DMA,
        ],
        compiler_params=pltpu.CompilerParams(use_tc_tiling_on_sc=False),
    )
    def gather_kernel(idx_hbm, table_hbm, out_hbm, idx_v, rows_v, sem):
        wid = lax.axis_index("s") * _NC + lax.axis_index("c")
        base = wid * _B_PER_W
        pltpu.sync_copy(idx_hbm.at[pl.ds(base, _B_PER_W)], idx_v)
        pltpu.async_copy(table_hbm.at[idx_v], rows_v, sem).wait()
        pltpu.sync_copy(rows_v, out_hbm.at[pl.ds(base, _B_PER_W)])

    return gather_kernel(data, emb)


_MB = 32
_NI = BATCH // _MB
_NBUF = 3


def _proj_kernel(x_ref, wt_ref, b_ref, out_hbm, buf, sems):
    j = pl.program_id(0)
    slot = lax.rem(j, _NBUF)

    @pl.when(j >= _NBUF)
    def _wait_slot():
        pltpu.make_async_copy(
            buf.at[slot],
            out_hbm.at[pl.ds((j - _NBUF) * _MB, _MB), :],
            sems.at[slot],
        ).wait()

    buf[slot] = lax.dot_general(
        x_ref[...], wt_ref[...],
        (((1,), (0,)), ((), ())),
        preferred_element_type=jnp.float32,
    ) + b_ref[...]

    pltpu.make_async_copy(
        buf.at[slot],
        out_hbm.at[pl.ds(j * _MB, _MB), :],
        sems.at[slot],
    ).start()

    @pl.when(j == _NI - 1)
    def _drain():
        for s in range(_NBUF):
            pltpu.make_async_copy(
                buf.at[s],
                out_hbm.at[pl.ds(0, _MB), :],
                sems.at[s],
            ).wait()


def _tc_project(x, Wt, b2):
    return pl.pallas_call(
        _proj_kernel,
        grid=(_NI,),
        in_specs=[
            pl.BlockSpec((_MB, N_HIDDEN), lambda j: (j, 0)),
            pl.BlockSpec((N_HIDDEN, N_FEATURES), lambda j: (0, 0)),
            pl.BlockSpec((1, N_FEATURES), lambda j: (0, 0)),
        ],
        out_specs=pl.BlockSpec(memory_space=pl.ANY),
        out_shape=jax.ShapeDtypeStruct((BATCH, N_FEATURES), jnp.float32),
        scratch_shapes=[
            pltpu.VMEM((_NBUF, _MB, N_FEATURES), jnp.float32),
            pltpu.SemaphoreType.DMA((_NBUF,)),
        ],
    )(x, Wt, b2)


def kernel(data, emb, W, b):
    x = _sc_gather(data, emb)
    return _tc_project(x, W.T, b[None, :])


# speedup vs baseline: 1.0075x; 1.0075x over previous
import functools

import jax
import jax.numpy as jnp
from jax import lax
from jax.experimental import pallas as pl
from jax.experimental.pallas import tpu as pltpu
from jax.experimental.pallas import tpu_sc as plsc

BATCH = 1024
N_HIDDEN = 16
N_FEATURES = 100000

_NC = 2
_NS = 16
_NW = _NC * _NS
_B_PER_W = BATCH // _NW


def _sc_gather(data, emb):
    mesh = plsc.VectorSubcoreMesh(core_axis_name="c", subcore_axis_name="s")

    @functools.partial(
        pl.kernel,
        mesh=mesh,
        out_type=jax.ShapeDtypeStruct((BATCH, N_HIDDEN), jnp.float32),
        scratch_types=[
            pltpu.VMEM((_B_PER_W,), jnp.int32),
            pltpu.VMEM((_B_PER_W, N_HIDDEN), jnp.float32),
            pltpu.SemaphoreType.DMA,
        ],
        compiler_params=pltpu.CompilerParams(use_tc_tiling_on_sc=False),
    )
    def gather_kernel(idx_hbm, table_hbm, out_hbm, idx_v, rows_v, sem):
        wid = lax.axis_index("s") * _NC + lax.axis_index("c")
        base = wid * _B_PER_W
        pltpu.sync_copy(idx_hbm.at[pl.ds(base, _B_PER_W)], idx_v)
        pltpu.async_copy(table_hbm.at[idx_v], rows_v, sem).wait()
        pltpu.sync_copy(rows_v, out_hbm.at[pl.ds(base, _B_PER_W)])

    return gather_kernel(data, emb)


_MB = 16
_NI = BATCH // _MB
_NBUF = 6


def _proj_kernel(x_ref, wt_ref, b_ref, out_hbm, buf, sems):
    j = pl.program_id(0)
    slot = lax.rem(j, _NBUF)

    @pl.when(j >= _NBUF)
    def _wait_slot():
        pltpu.make_async_copy(
            buf.at[slot],
            out_hbm.at[pl.ds((j - _NBUF) * _MB, _MB), :],
            sems.at[slot],
        ).wait()

    buf[slot] = lax.dot_general(
        x_ref[...], wt_ref[...],
        (((1,), (0,)), ((), ())),
        preferred_element_type=jnp.float32,
    ) + b_ref[...]

    pltpu.make_async_copy(
        buf.at[slot],
        out_hbm.at[pl.ds(j * _MB, _MB), :],
        sems.at[slot],
    ).start()

    @pl.when(j == _NI - 1)
    def _drain():
        for s in range(_NBUF):
            pltpu.make_async_copy(
                buf.at[s],
                out_hbm.at[pl.ds(0, _MB), :],
                sems.at[s],
            ).wait()


def _tc_project(x, Wt, b2):
    return pl.pallas_call(
        _proj_kernel,
        grid=(_NI,),
        in_specs=[
            pl.BlockSpec((_MB, N_HIDDEN), lambda j: (j, 0)),
            pl.BlockSpec((N_HIDDEN, N_FEATURES), lambda j: (0, 0)),
            pl.BlockSpec((1, N_FEATURES), lambda j: (0, 0)),
        ],
        out_specs=pl.BlockSpec(memory_space=pl.ANY),
        out_shape=jax.ShapeDtypeStruct((BATCH, N_FEATURES), jnp.float32),
        scratch_shapes=[
            pltpu.VMEM((_NBUF, _MB, N_FEATURES), jnp.float32),
            pltpu.SemaphoreType.DMA((_NBUF,)),
        ],
    )(x, Wt, b2)


def kernel(data, emb, W, b):
    x = _sc_gather(data, emb)
    return _tc_project(x, W.T, b[None, :])
